# baseline (speedup 1.0000x reference)
import functools

import jax
import jax.numpy as jnp
from jax import lax
from jax.experimental import pallas as pl
from jax.experimental.pallas import tpu as pltpu
from jax.experimental.pallas import tpu_sc as plsc

NC = 2
NS = 16
NW = NC * NS
CH = 128


def _sc_row_sums(pos_words, neg_words, embeddings):
    B = pos_words.shape[0]
    V, D = embeddings.shape
    assert D == 64 and B % (NW * CH) == 0
    bpw = B // NW
    nchunk = bpw // CH

    mesh = plsc.VectorSubcoreMesh(core_axis_name="c", subcore_axis_name="s")

    @functools.partial(
        pl.kernel,
        mesh=mesh,
        compiler_params=pltpu.CompilerParams(needs_layout_passes=False,
                                             use_tc_tiling_on_sc=False),
        out_type=[
            jax.ShapeDtypeStruct((B,), jnp.float32),
            jax.ShapeDtypeStruct((B,), jnp.float32),
        ],
        scratch_types=[
            pltpu.VMEM((nchunk, CH), jnp.int32),
            pltpu.VMEM((bpw, D), jnp.float32),
            pltpu.VMEM((bpw,), jnp.float32),
            pltpu.SemaphoreType.DMA,
        ],
    )
    def sc_kernel(pos_hbm, neg_hbm, emb_hbm, pos_out, neg_out,
                  idxs, rows, sums, sem):
        wid = lax.axis_index("s") * NC + lax.axis_index("c")
        base = wid * bpw

        def run(idx_hbm, out_hbm):
            for j in range(nchunk):
                pltpu.sync_copy(idx_hbm.at[pl.ds(base + j * CH, CH)],
                                idxs.at[j])
            copies = [
                pltpu.make_async_copy(emb_hbm.at[idxs.at[j]],
                                      rows.at[pl.ds(j * CH, CH)], sem)
                for j in range(nchunk)
            ]
            for c in copies:
                c.start()
            for c in copies:
                c.wait()

            lid = lax.iota(jnp.int32, 16)

            def s1(g, _):
                acc = jnp.zeros((16,), jnp.float32)
                for u in range(16):
                    r = g * 16 + u
                    v = (rows[r, pl.ds(0, 16)] + rows[r, pl.ds(16, 16)]
                         + rows[r, pl.ds(32, 16)] + rows[r, pl.ds(48, 16)])
                    acc = jnp.where(lid == u, jnp.sum(v), acc)
                sums[pl.ds(g * 16, 16)] = acc
                return _
            lax.fori_loop(0, bpw // 16, s1, None)

            pltpu.sync_copy(sums, out_hbm.at[pl.ds(base, bpw)])

        run(pos_hbm, pos_out)
        run(neg_hbm, neg_out)

    return sc_kernel(pos_words, neg_words, embeddings)


def _finish(pos_sums, neg_sums, batch):
    def body(p_ref, n_ref, o_ref):
        p = p_ref[...]
        n = n_ref[...]
        t = jnp.maximum(-p, 0.0) + jnp.log(1.0 + jnp.exp(-jnp.abs(p)))
        t = t + jnp.maximum(n, 0.0) + jnp.log(1.0 + jnp.exp(-jnp.abs(n)))
        o_ref[0, 0] = jnp.sum(t) * (1.0 / batch)

    out = pl.pallas_call(
        body,
        out_shape=jax.ShapeDtypeStruct((1, 1), jnp.float32),
        out_specs=pl.BlockSpec(memory_space=pltpu.SMEM),
    )(pos_sums, neg_sums)
    return out[0, 0]


def kernel(pos_words, neg_words, embeddings):
    B = pos_words.shape[0]
    pos_sums, neg_sums = _sc_row_sums(pos_words.astype(jnp.int32),
                                      neg_words.astype(jnp.int32),
                                      embeddings)
    return _finish(pos_sums.reshape(128, -1), neg_sums.reshape(128, -1), B)


# speedup vs baseline: 2.1500x; 2.1500x over previous
import functools

import jax
import jax.numpy as jnp
from jax import lax
from jax.experimental import pallas as pl
from jax.experimental.pallas import tpu as pltpu
from jax.experimental.pallas import tpu_sc as plsc

NC = 2
NS = 16
NW = NC * NS
CH = 32


def _sc_row_sums(pos_words, neg_words, emb3):
    B = pos_words.shape[0]
    NT, _, D = emb3.shape
    assert D == 64
    bpw = B // NW
    nchunk = 2 * bpw // CH

    mesh = plsc.VectorSubcoreMesh(core_axis_name="c", subcore_axis_name="s")

    @functools.partial(
        pl.kernel,
        mesh=mesh,
        compiler_params=pltpu.CompilerParams(needs_layout_passes=False,
                                             use_tc_tiling_on_sc=True),
        out_type=[
            jax.ShapeDtypeStruct((B,), jnp.float32),
            jax.ShapeDtypeStruct((B,), jnp.float32),
        ],
        scratch_types=[
            pltpu.VMEM((2 * bpw,), jnp.int32),
            pltpu.VMEM((CH, 8, D), jnp.float32),
            pltpu.VMEM((CH, 8, D), jnp.float32),
            pltpu.VMEM((2 * bpw,), jnp.float32),
            pltpu.SemaphoreType.DMA,
            pltpu.SemaphoreType.DMA,
        ],
    )
    def sc_kernel(pos_hbm, neg_hbm, emb_hbm, pos_out, neg_out,
                  idx_s, buf_a, buf_b, sums, sem_a, sem_b):
        wid = lax.axis_index("s") * NC + lax.axis_index("c")
        base = wid * bpw
        lid = lax.iota(jnp.int32, 16)

        for half, idx_hbm in ((0, pos_hbm), (1, neg_hbm)):
            pltpu.sync_copy(idx_hbm.at[pl.ds(base, bpw)],
                            idx_s.at[pl.ds(half * bpw, bpw)])

        def fire(k, buf, sem):
            for g in range(CH // 16):
                wv = idx_s[pl.ds(k * CH + g * 16, 16)] >> 3
                for u in range(16):
                    pltpu.make_async_copy(
                        emb_hbm.at[wv[u]], buf.at[g * 16 + u], sem).start()

        def drain(buf, sem):
            for u in range(CH):
                pltpu.make_async_copy(
                    emb_hbm.at[0], buf.at[u], sem).wait()

        def reduce_chunk(k, buf):
            for g in range(CH // 16):
                sv = idx_s[pl.ds(k * CH + g * 16, 16)] & 7
                acc = jnp.zeros((16,), jnp.float32)
                for u in range(16):
                    j = g * 16 + u
                    s = sv[u]
                    v = (buf[j, s, pl.ds(0, 16)] + buf[j, s, pl.ds(16, 16)]
                         + buf[j, s, pl.ds(32, 16)] + buf[j, s, pl.ds(48, 16)])
                    acc = jnp.where(lid == u, jnp.sum(v), acc)
                sums[pl.ds(k * CH + g * 16, 16)] = acc

        fire(0, buf_a, sem_a)
        fire(1, buf_b, sem_b)

        def body(i, _):
            k = i * 2
            drain(buf_a, sem_a)

            @pl.when(k + 2 < nchunk)
            def _():
                fire(k + 2, buf_a, sem_a)
            reduce_chunk(k, buf_a)

            drain(buf_b, sem_b)

            @pl.when(k + 3 < nchunk)
            def _():
                fire(k + 3, buf_b, sem_b)
            reduce_chunk(k + 1, buf_b)
            return _
        lax.fori_loop(0, nchunk // 2, body, None)

        pltpu.sync_copy(sums.at[pl.ds(0, bpw)], pos_out.at[pl.ds(base, bpw)])
        pltpu.sync_copy(sums.at[pl.ds(bpw, bpw)], neg_out.at[pl.ds(base, bpw)])

    return sc_kernel(pos_words, neg_words, emb3)


def _finish(pos_sums, neg_sums, batch):
    def body(p_ref, n_ref, o_ref):
        p = p_ref[...]
        n = n_ref[...]
        t = jnp.maximum(-p, 0.0) + jnp.log(1.0 + jnp.exp(-jnp.abs(p)))
        t = t + jnp.maximum(n, 0.0) + jnp.log(1.0 + jnp.exp(-jnp.abs(n)))
        o_ref[0, 0] = jnp.sum(t) * (1.0 / batch)

    out = pl.pallas_call(
        body,
        out_shape=jax.ShapeDtypeStruct((1, 1), jnp.float32),
        out_specs=pl.BlockSpec(memory_space=pltpu.SMEM),
    )(pos_sums, neg_sums)
    return out[0, 0]


def kernel(pos_words, neg_words, embeddings):
    B = pos_words.shape[0]
    V, D = embeddings.shape
    emb3 = embeddings.reshape(V // 8, 8, D)
    pos_sums, neg_sums = _sc_row_sums(pos_words.astype(jnp.int32),
                                      neg_words.astype(jnp.int32),
                                      emb3)
    return _finish(pos_sums.reshape(128, -1), neg_sums.reshape(128, -1), B)


# speedup vs baseline: 2.4922x; 1.1592x over previous
import functools

import jax
import jax.numpy as jnp
from jax import lax
from jax.experimental import pallas as pl
from jax.experimental.pallas import tpu as pltpu
from jax.experimental.pallas import tpu_sc as plsc

NC = 2
NS = 16
NW = NC * NS
CH = 32


def _sc_row_sums(pos_words, neg_words, emb3):
    B = pos_words.shape[0]
    NT, _, D = emb3.shape
    assert D == 64
    bpw = B // NW
    nchunk = 2 * bpw // CH

    mesh = plsc.VectorSubcoreMesh(core_axis_name="c", subcore_axis_name="s")

    @functools.partial(
        pl.kernel,
        mesh=mesh,
        compiler_params=pltpu.CompilerParams(needs_layout_passes=False,
                                             use_tc_tiling_on_sc=True),
        out_type=[
            jax.ShapeDtypeStruct((B,), jnp.float32),
            jax.ShapeDtypeStruct((B,), jnp.float32),
        ],
        scratch_types=[
            pltpu.VMEM((2 * bpw,), jnp.int32),
            pltpu.VMEM((CH, 8, D), jnp.float32),
            pltpu.VMEM((CH, 8, D), jnp.float32),
            pltpu.VMEM((2 * bpw,), jnp.float32),
            pltpu.SemaphoreType.DMA,
            pltpu.SemaphoreType.DMA,
        ],
    )
    def sc_kernel(pos_hbm, neg_hbm, emb_hbm, pos_out, neg_out,
                  idx_s, buf_a, buf_b, sums, sem_a, sem_b):
        wid = lax.axis_index("s") * NC + lax.axis_index("c")
        base = wid * bpw
        lid = lax.iota(jnp.int32, 16)

        for half, idx_hbm in ((0, pos_hbm), (1, neg_hbm)):
            pltpu.sync_copy(idx_hbm.at[pl.ds(base, bpw)],
                            idx_s.at[pl.ds(half * bpw, bpw)])

        def fire(k, buf, sem):
            for g in range(CH // 16):
                wv = idx_s[pl.ds(k * CH + g * 16, 16)]
                tv = wv >> 3
                sv = wv & 7
                for u in range(16):
                    pltpu.make_async_copy(
                        emb_hbm.at[tv[u], sv[u]],
                        buf.at[g * 16 + u, sv[u]], sem).start()

        def drain(buf, sem):
            for u in range(CH):
                pltpu.make_async_copy(
                    emb_hbm.at[0, 0], buf.at[u, 0], sem).wait()

        def reduce_chunk(k, buf):
            for g in range(CH // 16):
                sv = idx_s[pl.ds(k * CH + g * 16, 16)] & 7
                acc = jnp.zeros((16,), jnp.float32)
                for u in range(16):
                    j = g * 16 + u
                    s = sv[u]
                    v = (buf[j, s, pl.ds(0, 16)] + buf[j, s, pl.ds(16, 16)]
                         + buf[j, s, pl.ds(32, 16)] + buf[j, s, pl.ds(48, 16)])
                    acc = jnp.where(lid == u, jnp.sum(v), acc)
                sums[pl.ds(k * CH + g * 16, 16)] = acc

        fire(0, buf_a, sem_a)
        fire(1, buf_b, sem_b)

        def body(i, _):
            k = i * 2
            drain(buf_a, sem_a)

            @pl.when(k + 2 < nchunk)
            def _():
                fire(k + 2, buf_a, sem_a)
            reduce_chunk(k, buf_a)

            drain(buf_b, sem_b)

            @pl.when(k + 3 < nchunk)
            def _():
                fire(k + 3, buf_b, sem_b)
            reduce_chunk(k + 1, buf_b)
            return _
        lax.fori_loop(0, nchunk // 2, body, None)

        pltpu.sync_copy(sums.at[pl.ds(0, bpw)], pos_out.at[pl.ds(base, bpw)])
        pltpu.sync_copy(sums.at[pl.ds(bpw, bpw)], neg_out.at[pl.ds(base, bpw)])

    return sc_kernel(pos_words, neg_words, emb3)


def _finish(pos_sums, neg_sums, batch):
    def body(p_ref, n_ref, o_ref):
        p = p_ref[...]
        n = n_ref[...]
        t = jnp.maximum(-p, 0.0) + jnp.log(1.0 + jnp.exp(-jnp.abs(p)))
        t = t + jnp.maximum(n, 0.0) + jnp.log(1.0 + jnp.exp(-jnp.abs(n)))
        o_ref[0, 0] = jnp.sum(t) * (1.0 / batch)

    out = pl.pallas_call(
        body,
        out_shape=jax.ShapeDtypeStruct((1, 1), jnp.float32),
        out_specs=pl.BlockSpec(memory_space=pltpu.SMEM),
    )(pos_sums, neg_sums)
    return out[0, 0]


def kernel(pos_words, neg_words, embeddings):
    B = pos_words.shape[0]
    V, D = embeddings.shape
    emb3 = embeddings.reshape(V // 8, 8, D)
    pos_sums, neg_sums = _sc_row_sums(pos_words.astype(jnp.int32),
                                      neg_words.astype(jnp.int32),
                                      emb3)
    return _finish(pos_sums.reshape(128, -1), neg_sums.reshape(128, -1), B)


# speedup vs baseline: 2.4956x; 1.0013x over previous
import functools

import jax
import jax.numpy as jnp
from jax import lax
from jax.experimental import pallas as pl
from jax.experimental.pallas import tpu as pltpu
from jax.experimental.pallas import tpu_sc as plsc

NC = 2
NS = 16
NW = NC * NS
CH = 32


def _sc_row_sums(pos_words, neg_words, emb3):
    B = pos_words.shape[0]
    NT, _, D = emb3.shape
    assert D == 64
    bpw = B // NW
    nchunk = 2 * bpw // CH

    mesh = plsc.VectorSubcoreMesh(core_axis_name="c", subcore_axis_name="s")

    @functools.partial(
        pl.kernel,
        mesh=mesh,
        compiler_params=pltpu.CompilerParams(needs_layout_passes=False,
                                             use_tc_tiling_on_sc=True),
        out_type=[
            jax.ShapeDtypeStruct((B,), jnp.float32),
            jax.ShapeDtypeStruct((B,), jnp.float32),
        ],
        scratch_types=[
            pltpu.VMEM((2 * bpw,), jnp.int32),
            pltpu.VMEM((CH, 8, D), jnp.float32),
            pltpu.VMEM((CH, 8, D), jnp.float32),
            pltpu.VMEM((2 * bpw,), jnp.float32),
            pltpu.SemaphoreType.DMA((4,)),
            pltpu.SemaphoreType.DMA((4,)),
        ],
    )
    def sc_kernel(pos_hbm, neg_hbm, emb_hbm, pos_out, neg_out,
                  idx_s, buf_a, buf_b, sums, sem_a, sem_b):
        wid = lax.axis_index("s") * NC + lax.axis_index("c")
        base = wid * bpw
        lid = lax.iota(jnp.int32, 16)

        for half, idx_hbm in ((0, pos_hbm), (1, neg_hbm)):
            pltpu.sync_copy(idx_hbm.at[pl.ds(base, bpw)],
                            idx_s.at[pl.ds(half * bpw, bpw)])

        def fire(k, buf, sem):
            for g in range(CH // 16):
                wv = idx_s[pl.ds(k * CH + g * 16, 16)]
                tv = wv >> 3
                sv = wv & 7
                for u in range(16):
                    pltpu.make_async_copy(
                        emb_hbm.at[tv[u], sv[u]],
                        buf.at[g * 16 + u, sv[u]], sem.at[u % 4]).start()

        def drain(buf, sem):
            for u in range(CH):
                pltpu.make_async_copy(
                    emb_hbm.at[0, 0], buf.at[u, 0], sem.at[u % 4]).wait()

        def reduce_chunk(k, buf):
            for g in range(CH // 16):
                sv = idx_s[pl.ds(k * CH + g * 16, 16)] & 7
                acc = jnp.zeros((16,), jnp.float32)
                for u in range(16):
                    j = g * 16 + u
                    s = sv[u]
                    v = (buf[j, s, pl.ds(0, 16)] + buf[j, s, pl.ds(16, 16)]
                         + buf[j, s, pl.ds(32, 16)] + buf[j, s, pl.ds(48, 16)])
                    acc = jnp.where(lid == u, jnp.sum(v), acc)
                sums[pl.ds(k * CH + g * 16, 16)] = acc

        fire(0, buf_a, sem_a)
        fire(1, buf_b, sem_b)

        def body(i, _):
            k = i * 2
            drain(buf_a, sem_a)

            @pl.when(k + 2 < nchunk)
            def _():
                fire(k + 2, buf_a, sem_a)
            reduce_chunk(k, buf_a)

            drain(buf_b, sem_b)

            @pl.when(k + 3 < nchunk)
            def _():
                fire(k + 3, buf_b, sem_b)
            reduce_chunk(k + 1, buf_b)
            return _
        lax.fori_loop(0, nchunk // 2, body, None)

        pltpu.sync_copy(sums.at[pl.ds(0, bpw)], pos_out.at[pl.ds(base, bpw)])
        pltpu.sync_copy(sums.at[pl.ds(bpw, bpw)], neg_out.at[pl.ds(base, bpw)])

    return sc_kernel(pos_words, neg_words, emb3)


def _finish(pos_sums, neg_sums, batch):
    def body(p_ref, n_ref, o_ref):
        p = p_ref[...]
        n = n_ref[...]
        t = jnp.maximum(-p, 0.0) + jnp.log(1.0 + jnp.exp(-jnp.abs(p)))
        t = t + jnp.maximum(n, 0.0) + jnp.log(1.0 + jnp.exp(-jnp.abs(n)))
        o_ref[0, 0] = jnp.sum(t) * (1.0 / batch)

    out = pl.pallas_call(
        body,
        out_shape=jax.ShapeDtypeStruct((1, 1), jnp.float32),
        out_specs=pl.BlockSpec(memory_space=pltpu.SMEM),
    )(pos_sums, neg_sums)
    return out[0, 0]


def kernel(pos_words, neg_words, embeddings):
    B = pos_words.shape[0]
    V, D = embeddings.shape
    emb3 = embeddings.reshape(V // 8, 8, D)
    pos_sums, neg_sums = _sc_row_sums(pos_words.astype(jnp.int32),
                                      neg_words.astype(jnp.int32),
                                      emb3)
    return _finish(pos_sums.reshape(128, -1), neg_sums.reshape(128, -1), B)
